# baseline (speedup 1.0000x reference)
import functools

import jax
import jax.numpy as jnp
from jax import lax
from jax.experimental import pallas as pl
from jax.experimental.pallas import tpu as pltpu
from jax.experimental.pallas import tpu_sc as plsc

_B = 4096
_S = 200
_D = 32
_ST = _S // 8
_BT = _B // 128
_NW = 32
_G = _ST * _BT * 2
_PER_W = _G // _NW
_ROWS = 4 * 128
_CP = 2 * _D * 2 + 1

_mesh = plsc.VectorSubcoreMesh(core_axis_name="c", subcore_axis_name="s")


@functools.partial(
    pl.kernel,
    mesh=_mesh,
    compiler_params=pltpu.CompilerParams(use_tc_tiling_on_sc=False,
                                         needs_layout_passes=False),
    out_type=jax.ShapeDtypeStruct((_S, 2 * _D, _BT, 128), jnp.float32),
    scratch_types=[
        pltpu.VMEM((2, 4, 128), jnp.int32),
        pltpu.VMEM((2, 4, 128), jnp.int32),
        pltpu.VMEM((2, _ROWS, _D), jnp.float32),
        pltpu.VMEM((2, _ROWS, _D), jnp.float32),
        pltpu.VMEM((4, 2 * _D, _CP), jnp.float32),
        pltpu.SemaphoreType.DMA,
        pltpu.SemaphoreType.DMA,
        pltpu.SemaphoreType.DMA,
        pltpu.SemaphoreType.DMA,
        pltpu.SemaphoreType.DMA,
    ],
)
def _embed_gather(sign_hbm, stroke_hbm, sid_hbm, tid_hbm, out_hbm,
                  sidx_v, tidx_v, srows_v, trows_v, comb_v,
                  isem0, isem1, gsem0, gsem1, osem):
    wid = lax.axis_index("s") * 2 + lax.axis_index("c")
    g0 = wid * _PER_W
    isem = (isem0, isem1)
    gsem = (gsem0, gsem1)

    def coords(k):
        g = g0 + k
        st = g >> 6
        bt = (g >> 1) & 31
        h = g & 1
        return st, bt, h

    def issue_idx(k, b):
        st, bt, h = coords(k)
        pltpu.async_copy(sid_hbm.at[st, bt, pl.ds(h * 4, 4)], sidx_v.at[b], isem[b])
        pltpu.async_copy(tid_hbm.at[st, bt, pl.ds(h * 4, 4)], tidx_v.at[b], isem[b])

    def wait_idx(b):
        pltpu.make_async_copy(sid_hbm.at[0, 0, pl.ds(0, 4)], sidx_v.at[b], isem[b]).wait()
        pltpu.make_async_copy(tid_hbm.at[0, 0, pl.ds(0, 4)], tidx_v.at[b], isem[b]).wait()

    def issue_gathers(b):
        for si in range(4):
            pltpu.async_copy(sign_hbm.at[sidx_v.at[b, si]],
                             srows_v.at[b, pl.ds(si * 128, 128)], gsem[b])
            pltpu.async_copy(stroke_hbm.at[tidx_v.at[b, si]],
                             trows_v.at[b, pl.ds(si * 128, 128)], gsem[b])

    def wait_gathers(b):
        for si in range(4):
            pltpu.make_async_copy(sign_hbm.at[sidx_v.at[b, si]],
                                  srows_v.at[b, pl.ds(si * 128, 128)], gsem[b]).wait()
            pltpu.make_async_copy(stroke_hbm.at[tidx_v.at[b, si]],
                                  trows_v.at[b, pl.ds(si * 128, 128)], gsem[b]).wait()

    def out_slice(k):
        st, bt, h = coords(k)
        return out_hbm.at[pl.ds(st * 8 + h * 4, 4), :, bt]

    def issue_out(k):
        pltpu.async_copy(comb_v.at[:, :, pl.ds(0, 128)], out_slice(k), osem)

    def wait_out(k):
        pltpu.make_async_copy(comb_v.at[:, :, pl.ds(0, 128)], out_slice(k), osem).wait()

    lanes = lax.iota(jnp.int32, 16)
    dvecs = [c * 16 + lanes for c in range(4)]

    def transpose(b):
        @plsc.parallel_loop(0, 4)
        def si_body(si):
            comb_si = comb_v.at[si]

            @plsc.parallel_loop(0, 8, unroll=2)
            def g_body(g):
                t0 = si * 128 + g * 16
                for j in range(16):
                    t = t0 + j
                    bivec = jnp.full((16,), g * 16 + j, jnp.int32)
                    for c in range(2):
                        vals = srows_v[b, t, pl.ds(c * 16, 16)]
                        plsc.store_scatter(comb_si, [dvecs[c], bivec], vals)
                    for c in range(2):
                        vals = trows_v[b, t, pl.ds(c * 16, 16)]
                        plsc.store_scatter(comb_si, [dvecs[2 + c], bivec], vals)

    def step(k, b, first, last):
        if not last:
            wait_idx(1 - b)
            issue_gathers(1 - b)
        wait_gathers(b)
        if not first:
            wait_out(k - 1)
        transpose(b)
        issue_out(k)
        if not last:
            @pl.when(k + 2 <= _PER_W - 1)
            def _():
                issue_idx(k + 2, b)

    issue_idx(0, 0)
    issue_idx(1, 1)
    wait_idx(0)
    issue_gathers(0)
    step(0, 0, first=True, last=False)

    def loop_body(i, _):
        k = 1 + 2 * i
        step(k, 1, first=False, last=False)
        step(k + 1, 0, first=False, last=False)
        return 0

    lax.fori_loop(0, (_PER_W - 2) // 2, loop_body, 0)

    step(_PER_W - 1, 1, first=False, last=True)
    wait_out(_PER_W - 1)


def _tile_decompose_ids(ids):
    return ids.T.reshape(_ST, 8, _BT, 128).transpose(0, 2, 1, 3)


def kernel(sign_table, stroke_table, sign_ids, stroke_ids):
    out4 = _embed_gather(sign_table, stroke_table,
                         _tile_decompose_ids(sign_ids),
                         _tile_decompose_ids(stroke_ids))
    return jnp.transpose(out4, (2, 3, 0, 1)).reshape(_B, _S, 2 * _D)


# speedup vs baseline: 1.0097x; 1.0097x over previous
import functools

import jax
import jax.numpy as jnp
from jax import lax
from jax.experimental import pallas as pl
from jax.experimental.pallas import tpu as pltpu
from jax.experimental.pallas import tpu_sc as plsc

_B = 4096
_S = 200
_D = 32
_ST = _S // 8
_BT = _B // 128
_NW = 32
_G = _ST * _BT * 2
_PER_W = _G // _NW
_ROWS = 4 * 128
_CP = 2 * _D * 2 + 1

_mesh = plsc.VectorSubcoreMesh(core_axis_name="c", subcore_axis_name="s")


@functools.partial(
    pl.kernel,
    mesh=_mesh,
    compiler_params=pltpu.CompilerParams(use_tc_tiling_on_sc=False,
                                         needs_layout_passes=False),
    out_type=jax.ShapeDtypeStruct((_S, 2 * _D, _BT, 128), jnp.float32),
    scratch_types=[
        pltpu.VMEM((2, 4, 128), jnp.int32),
        pltpu.VMEM((2, 4, 128), jnp.int32),
        pltpu.VMEM((2, _ROWS, _D), jnp.float32),
        pltpu.VMEM((2, _ROWS, _D), jnp.float32),
        pltpu.VMEM((4, 2 * _D, _CP), jnp.float32),
        pltpu.SemaphoreType.DMA,
        pltpu.SemaphoreType.DMA,
        pltpu.SemaphoreType.DMA,
        pltpu.SemaphoreType.DMA,
        pltpu.SemaphoreType.DMA,
    ],
)
def _embed_gather(sign_hbm, stroke_hbm, sid_hbm, tid_hbm, out_hbm,
                  sidx_v, tidx_v, srows_v, trows_v, comb_v,
                  isem0, isem1, gsem0, gsem1, osem):
    wid = lax.axis_index("s") * 2 + lax.axis_index("c")
    g0 = wid * _PER_W
    isem = (isem0, isem1)
    gsem = (gsem0, gsem1)

    def coords(k):
        g = g0 + k
        st = g >> 6
        bt = (g >> 1) & 31
        h = g & 1
        return st, bt, h

    def issue_idx(k, b):
        st, bt, h = coords(k)
        pltpu.async_copy(sid_hbm.at[st, bt, pl.ds(h * 4, 4)], sidx_v.at[b], isem[b])
        pltpu.async_copy(tid_hbm.at[st, bt, pl.ds(h * 4, 4)], tidx_v.at[b], isem[b])

    def wait_idx(b):
        pltpu.make_async_copy(sid_hbm.at[0, 0, pl.ds(0, 4)], sidx_v.at[b], isem[b]).wait()
        pltpu.make_async_copy(tid_hbm.at[0, 0, pl.ds(0, 4)], tidx_v.at[b], isem[b]).wait()

    def issue_gathers(b):
        for si in range(4):
            pltpu.async_copy(sign_hbm.at[sidx_v.at[b, si]],
                             srows_v.at[b, pl.ds(si * 128, 128)], gsem[b])
            pltpu.async_copy(stroke_hbm.at[tidx_v.at[b, si]],
                             trows_v.at[b, pl.ds(si * 128, 128)], gsem[b])

    def wait_gathers(b):
        for si in range(4):
            pltpu.make_async_copy(sign_hbm.at[sidx_v.at[b, si]],
                                  srows_v.at[b, pl.ds(si * 128, 128)], gsem[b]).wait()
            pltpu.make_async_copy(stroke_hbm.at[tidx_v.at[b, si]],
                                  trows_v.at[b, pl.ds(si * 128, 128)], gsem[b]).wait()

    def out_slice(k):
        st, bt, h = coords(k)
        return out_hbm.at[pl.ds(st * 8 + h * 4, 4), :, bt]

    def issue_out(k):
        pltpu.async_copy(comb_v.at[:, :, pl.ds(0, 128)], out_slice(k), osem)

    def wait_out(k):
        pltpu.make_async_copy(comb_v.at[:, :, pl.ds(0, 128)], out_slice(k), osem).wait()

    lanes = lax.iota(jnp.int32, 16)
    dvecs = [c * 16 + lanes for c in range(4)]

    def transpose(b):
        def si_body(si, _):
            comb_si = comb_v.at[si]

            @plsc.parallel_loop(0, 8, unroll=2)
            def g_body(g):
                t0 = si * 128 + g * 16
                for j in range(16):
                    t = t0 + j
                    bivec = jnp.full((16,), g * 16 + j, jnp.int32)
                    for c in range(2):
                        vals = srows_v[b, t, pl.ds(c * 16, 16)]
                        plsc.store_scatter(comb_si, [dvecs[c], bivec], vals)
                    for c in range(2):
                        vals = trows_v[b, t, pl.ds(c * 16, 16)]
                        plsc.store_scatter(comb_si, [dvecs[2 + c], bivec], vals)

            return 0

        lax.fori_loop(0, 4, si_body, 0)

    def step(k, b, first, last):
        if not last:
            wait_idx(1 - b)
            issue_gathers(1 - b)
        wait_gathers(b)
        if not first:
            wait_out(k - 1)
        transpose(b)
        issue_out(k)
        if not last:
            @pl.when(k + 2 <= _PER_W - 1)
            def _():
                issue_idx(k + 2, b)

    issue_idx(0, 0)
    issue_idx(1, 1)
    wait_idx(0)
    issue_gathers(0)
    step(0, 0, first=True, last=False)

    def loop_body(i, _):
        k = 1 + 2 * i
        step(k, 1, first=False, last=False)
        step(k + 1, 0, first=False, last=False)
        return 0

    lax.fori_loop(0, (_PER_W - 2) // 2, loop_body, 0)

    step(_PER_W - 1, 1, first=False, last=True)
    wait_out(_PER_W - 1)


def _tile_decompose_ids(ids):
    return ids.T.reshape(_ST, 8, _BT, 128).transpose(0, 2, 1, 3)


def kernel(sign_table, stroke_table, sign_ids, stroke_ids):
    out4 = _embed_gather(sign_table, stroke_table,
                         _tile_decompose_ids(sign_ids),
                         _tile_decompose_ids(stroke_ids))
    return jnp.transpose(out4, (2, 3, 0, 1)).reshape(_B, _S, 2 * _D)


# speedup vs baseline: 1.5616x; 1.5466x over previous
import functools

import jax
import jax.numpy as jnp
from jax import lax
from jax.experimental import pallas as pl
from jax.experimental.pallas import tpu as pltpu
from jax.experimental.pallas import tpu_sc as plsc

_B = 4096
_S = 200
_D = 32
_ST = _S // 8
_BT = _B // 128
_NW = 32
_G = _ST * _BT * 2
_PER_W = _G // _NW
_ROWS = 4 * 128
_CP = 2 * _D * 2 + 1

_mesh = plsc.VectorSubcoreMesh(core_axis_name="c", subcore_axis_name="s")


@functools.partial(
    pl.kernel,
    mesh=_mesh,
    compiler_params=pltpu.CompilerParams(use_tc_tiling_on_sc=False,
                                         needs_layout_passes=False),
    out_type=jax.ShapeDtypeStruct((_S, 8, _BT, 8, 128), jnp.float32),
    scratch_types=[
        pltpu.VMEM((2, 4, 128), jnp.int32),
        pltpu.VMEM((2, 4, 128), jnp.int32),
        pltpu.VMEM((2, _ROWS, _D), jnp.float32),
        pltpu.VMEM((2, _ROWS, _D), jnp.float32),
        pltpu.VMEM((4, 8, 8, _CP), jnp.float32),
        pltpu.SemaphoreType.DMA,
        pltpu.SemaphoreType.DMA,
        pltpu.SemaphoreType.DMA,
        pltpu.SemaphoreType.DMA,
        pltpu.SemaphoreType.DMA,
    ],
)
def _embed_gather(sign_hbm, stroke_hbm, sid_hbm, tid_hbm, out_hbm,
                  sidx_v, tidx_v, srows_v, trows_v, comb_v,
                  isem0, isem1, gsem0, gsem1, osem):
    wid = lax.axis_index("s") * 2 + lax.axis_index("c")
    g0 = wid * _PER_W
    isem = (isem0, isem1)
    gsem = (gsem0, gsem1)

    def coords(k):
        g = g0 + k
        st = g >> 6
        bt = (g >> 1) & 31
        h = g & 1
        return st, bt, h

    def issue_idx(k, b):
        st, bt, h = coords(k)
        pltpu.async_copy(sid_hbm.at[st, bt, pl.ds(h * 4, 4)], sidx_v.at[b], isem[b])
        pltpu.async_copy(tid_hbm.at[st, bt, pl.ds(h * 4, 4)], tidx_v.at[b], isem[b])

    def wait_idx(b):
        pltpu.make_async_copy(sid_hbm.at[0, 0, pl.ds(0, 4)], sidx_v.at[b], isem[b]).wait()
        pltpu.make_async_copy(tid_hbm.at[0, 0, pl.ds(0, 4)], tidx_v.at[b], isem[b]).wait()

    def issue_gathers(b):
        for si in range(4):
            pltpu.async_copy(sign_hbm.at[sidx_v.at[b, si]],
                             srows_v.at[b, pl.ds(si * 128, 128)], gsem[b])
            pltpu.async_copy(stroke_hbm.at[tidx_v.at[b, si]],
                             trows_v.at[b, pl.ds(si * 128, 128)], gsem[b])

    def wait_gathers(b):
        for si in range(4):
            pltpu.make_async_copy(sign_hbm.at[sidx_v.at[b, si]],
                                  srows_v.at[b, pl.ds(si * 128, 128)], gsem[b]).wait()
            pltpu.make_async_copy(stroke_hbm.at[tidx_v.at[b, si]],
                                  trows_v.at[b, pl.ds(si * 128, 128)], gsem[b]).wait()

    def out_slice(k):
        st, bt, h = coords(k)
        return out_hbm.at[pl.ds(st * 8 + h * 4, 4), :, bt]

    def issue_out(k):
        pltpu.async_copy(comb_v.at[:, :, :, pl.ds(0, 128)], out_slice(k), osem)

    def wait_out(k):
        pltpu.make_async_copy(comb_v.at[:, :, :, pl.ds(0, 128)], out_slice(k),
                              osem).wait()

    lanes = lax.iota(jnp.int32, 16)
    dtvecs = [(c * 16 + lanes) >> 3 for c in range(4)]
    divecs = [(c * 16 + lanes) & 7 for c in range(4)]

    def transpose(b):
        def si_body(si, _):
            comb_si = comb_v.at[si]

            @plsc.parallel_loop(0, 8, unroll=2)
            def g_body(g):
                t0 = si * 128 + g * 16
                for j in range(16):
                    t = t0 + j
                    bivec = jnp.full((16,), g * 16 + j, jnp.int32)
                    for c in range(2):
                        vals = srows_v[b, t, pl.ds(c * 16, 16)]
                        plsc.store_scatter(comb_si,
                                           [dtvecs[c], divecs[c], bivec], vals)
                    for c in range(2):
                        vals = trows_v[b, t, pl.ds(c * 16, 16)]
                        plsc.store_scatter(
                            comb_si, [dtvecs[2 + c], divecs[2 + c], bivec], vals)

            return 0

        lax.fori_loop(0, 4, si_body, 0)

    def step(k, b, first, last):
        if not last:
            wait_idx(1 - b)
            issue_gathers(1 - b)
        wait_gathers(b)
        if not first:
            wait_out(k - 1)
        transpose(b)
        issue_out(k)
        if not last:
            @pl.when(k + 2 <= _PER_W - 1)
            def _():
                issue_idx(k + 2, b)

    issue_idx(0, 0)
    issue_idx(1, 1)
    wait_idx(0)
    issue_gathers(0)
    step(0, 0, first=True, last=False)

    def loop_body(i, _):
        k = 1 + 2 * i
        step(k, 1, first=False, last=False)
        step(k + 1, 0, first=False, last=False)
        return 0

    lax.fori_loop(0, (_PER_W - 2) // 2, loop_body, 0)

    step(_PER_W - 1, 1, first=False, last=True)
    wait_out(_PER_W - 1)


def _tile_decompose_ids(ids):
    return ids.T.reshape(_ST, 8, _BT, 128).transpose(0, 2, 1, 3)


def kernel(sign_table, stroke_table, sign_ids, stroke_ids):
    out5 = _embed_gather(sign_table, stroke_table,
                         _tile_decompose_ids(sign_ids),
                         _tile_decompose_ids(stroke_ids))
    return jnp.transpose(out5, (2, 4, 0, 1, 3)).reshape(_B, _S, 2 * _D)


# speedup vs baseline: 1.6645x; 1.0659x over previous
import functools

import jax
import jax.numpy as jnp
from jax import lax
from jax.experimental import pallas as pl
from jax.experimental.pallas import tpu as pltpu
from jax.experimental.pallas import tpu_sc as plsc

_B = 4096
_S = 200
_D = 32
_ST = _S // 8
_BT = _B // 128
_NW = 32
_G = _ST * _BT * 2
_PER_W = _G // _NW
_ROWS = 4 * 128
_CP = 2 * _D * 2 + 1

_mesh = plsc.VectorSubcoreMesh(core_axis_name="c", subcore_axis_name="s")


@functools.partial(
    pl.kernel,
    mesh=_mesh,
    compiler_params=pltpu.CompilerParams(use_tc_tiling_on_sc=False,
                                         needs_layout_passes=False),
    out_type=jax.ShapeDtypeStruct((_S, 8, _BT, 8, 128), jnp.float32),
    scratch_types=[
        pltpu.VMEM((2, 4, 128), jnp.int32),
        pltpu.VMEM((2, 4, 128), jnp.int32),
        pltpu.VMEM((2, _ROWS, _D), jnp.float32),
        pltpu.VMEM((2, _ROWS, _D), jnp.float32),
        pltpu.VMEM((2, 2, 8, 8, _CP), jnp.float32),
        pltpu.SemaphoreType.DMA,
        pltpu.SemaphoreType.DMA,
        pltpu.SemaphoreType.DMA,
        pltpu.SemaphoreType.DMA,
        pltpu.SemaphoreType.DMA,
        pltpu.SemaphoreType.DMA,
    ],
)
def _embed_gather(sign_hbm, stroke_hbm, sid_hbm, tid_hbm, out_hbm,
                  sidx_v, tidx_v, srows_v, trows_v, comb_v,
                  isem0, isem1, gsem0, gsem1, osem0, osem1):
    wid = lax.axis_index("s") * 2 + lax.axis_index("c")
    g0 = wid * _PER_W
    isem = (isem0, isem1)
    gsem = (gsem0, gsem1)
    osem = (osem0, osem1)

    def coords(k):
        g = g0 + k
        st = g >> 6
        bt = (g >> 1) & 31
        h = g & 1
        return st, bt, h

    def issue_idx(k, b):
        st, bt, h = coords(k)
        pltpu.async_copy(sid_hbm.at[st, bt, pl.ds(h * 4, 4)], sidx_v.at[b], isem[b])
        pltpu.async_copy(tid_hbm.at[st, bt, pl.ds(h * 4, 4)], tidx_v.at[b], isem[b])

    def wait_idx(b):
        pltpu.make_async_copy(sid_hbm.at[0, 0, pl.ds(0, 4)], sidx_v.at[b], isem[b]).wait()
        pltpu.make_async_copy(tid_hbm.at[0, 0, pl.ds(0, 4)], tidx_v.at[b], isem[b]).wait()

    def issue_gathers(b):
        for si in range(4):
            pltpu.async_copy(sign_hbm.at[sidx_v.at[b, si]],
                             srows_v.at[b, pl.ds(si * 128, 128)], gsem[b])
            pltpu.async_copy(stroke_hbm.at[tidx_v.at[b, si]],
                             trows_v.at[b, pl.ds(si * 128, 128)], gsem[b])

    def wait_gathers(b):
        for si in range(4):
            pltpu.make_async_copy(sign_hbm.at[sidx_v.at[b, si]],
                                  srows_v.at[b, pl.ds(si * 128, 128)], gsem[b]).wait()
            pltpu.make_async_copy(stroke_hbm.at[tidx_v.at[b, si]],
                                  trows_v.at[b, pl.ds(si * 128, 128)], gsem[b]).wait()

    def out_slice(k, h2):
        st, bt, h = coords(k)
        return out_hbm.at[pl.ds(st * 8 + h * 4 + h2 * 2, 2), :, bt]

    def issue_out(k, h2):
        pltpu.async_copy(comb_v.at[h2, :, :, :, pl.ds(0, 128)],
                         out_slice(k, h2), osem[h2])

    def wait_out(k, h2):
        pltpu.make_async_copy(comb_v.at[h2, :, :, :, pl.ds(0, 128)],
                              out_slice(k, h2), osem[h2]).wait()

    lanes = lax.iota(jnp.int32, 16)
    dtvecs = [(c * 16 + lanes) >> 3 for c in range(4)]
    divecs = [(c * 16 + lanes) & 7 for c in range(4)]

    def transpose_half(b, h2):
        def si_body(i, _):
            si = h2 * 2 + i
            comb_si = comb_v.at[h2, i]

            @plsc.parallel_loop(0, 8, unroll=2)
            def g_body(g):
                t0 = si * 128 + g * 16
                for j in range(16):
                    t = t0 + j
                    bivec = jnp.full((16,), g * 16 + j, jnp.int32)
                    for c in range(2):
                        vals = srows_v[b, t, pl.ds(c * 16, 16)]
                        plsc.store_scatter(comb_si,
                                           [dtvecs[c], divecs[c], bivec], vals)
                    for c in range(2):
                        vals = trows_v[b, t, pl.ds(c * 16, 16)]
                        plsc.store_scatter(
                            comb_si, [dtvecs[2 + c], divecs[2 + c], bivec], vals)

            return 0

        lax.fori_loop(0, 2, si_body, 0)

    def step(k, b, first, last):
        if not last:
            wait_idx(1 - b)
            issue_gathers(1 - b)
        wait_gathers(b)
        for h2 in range(2):
            if not first:
                wait_out(k - 1, h2)
            transpose_half(b, h2)
            issue_out(k, h2)
        if not last:
            @pl.when(k + 2 <= _PER_W - 1)
            def _():
                issue_idx(k + 2, b)

    issue_idx(0, 0)
    issue_idx(1, 1)
    wait_idx(0)
    issue_gathers(0)
    step(0, 0, first=True, last=False)

    def loop_body(i, _):
        k = 1 + 2 * i
        step(k, 1, first=False, last=False)
        step(k + 1, 0, first=False, last=False)
        return 0

    lax.fori_loop(0, (_PER_W - 2) // 2, loop_body, 0)

    step(_PER_W - 1, 1, first=False, last=True)
    wait_out(_PER_W - 1, 0)
    wait_out(_PER_W - 1, 1)


def _tile_decompose_ids(ids):
    return ids.T.reshape(_ST, 8, _BT, 128).transpose(0, 2, 1, 3)


def kernel(sign_table, stroke_table, sign_ids, stroke_ids):
    out5 = _embed_gather(sign_table, stroke_table,
                         _tile_decompose_ids(sign_ids),
                         _tile_decompose_ids(stroke_ids))
    return jnp.transpose(out5, (2, 4, 0, 1, 3)).reshape(_B, _S, 2 * _D)
